# Initial kernel scaffold; baseline (speedup 1.0000x reference)
import jax
import jax.numpy as jnp
from jax.experimental import pallas as pl


def kernel(x_gen, x_true, node_seg, q_gen, q_true, edge_seg):
    raise NotImplementedError("write your pallas kernel here")


# speedup vs baseline: 40.1475x; 40.1475x over previous
import functools

import jax
import jax.numpy as jnp
from jax import lax
from jax.experimental import pallas as pl
from jax.experimental.pallas import tpu as pltpu
from jax.experimental.pallas import tpu_sc as plsc

NUM_SEG = 256
N_NODES = 100000
N_EDGES = 6400000

NC = 2
NS = 16
L = 16
NW = NC * NS

EDGES_PER_TILE = N_EDGES // NW
ECHUNK = 10000
N_ECHUNK = EDGES_PER_TILE // ECHUNK
EUNROLL = 5
EVEC_ITERS = ECHUNK // (L * EUNROLL)

NODES_PAD = 102400
NODES_PER_TILE = NODES_PAD // NW
NUNROLL = 5
NVEC_ITERS = NODES_PER_TILE // (L * NUNROLL)

TAB_ROWS = 264


def _lane_iota():
    return lax.iota(jnp.int32, L)


def _reduce_table(tab, partials, base):
    lane = _lane_iota()

    def outer(g, _):
        def inner(s2, vec):
            row = tab[pl.ds((g * L + s2) * L, L)]
            scal = jnp.sum(row)
            return jnp.where(lane == s2, scal, vec)

        vec = lax.fori_loop(0, L, inner, jnp.zeros((L,), jnp.float32))
        partials[pl.ds(base + g * L, L)] = vec
        return _

    lax.fori_loop(0, NUM_SEG // L, outer, 0)


def _sc_body(qg_hbm, qt_hbm, es_hbm, xg_hbm, xt_hbm, ns_hbm, out_hbm,
             qg_buf, qt_buf, es_buf, xg_buf, xt_buf, ns_buf,
             etab, ntab, ctab, partials):
    wid = lax.axis_index("s") * NC + lax.axis_index("c")
    lane = _lane_iota()
    ones = jnp.ones((L,), jnp.float32)

    def zero_body(r, _):
        z = jnp.zeros((L,), jnp.float32)
        sl = pl.ds(r * L, L)
        etab[sl] = z
        ntab[sl] = z
        ctab[sl] = z
        return _

    lax.fori_loop(0, TAB_ROWS, zero_body, 0)

    nbase = wid * NODES_PER_TILE
    pltpu.sync_copy(xg_hbm.at[:, pl.ds(nbase, NODES_PER_TILE)], xg_buf)
    pltpu.sync_copy(xt_hbm.at[:, pl.ds(nbase, NODES_PER_TILE)], xt_buf)
    pltpu.sync_copy(ns_hbm.at[pl.ds(nbase, NODES_PER_TILE)], ns_buf)

    def node_body(j, _):
        for u in range(NUNROLL):
            sl = pl.ds((j * NUNROLL + u) * L, L)
            dx = xg_buf[0, sl] - xt_buf[0, sl]
            dy = xg_buf[1, sl] - xt_buf[1, sl]
            dz = xg_buf[2, sl] - xt_buf[2, sl]
            err = dx * dx + dy * dy + dz * dz
            idx = ns_buf[sl] * L + lane
            plsc.addupdate_scatter(ntab, [idx], err)
            plsc.addupdate_scatter(ctab, [idx], ones)
        return _

    lax.fori_loop(0, NVEC_ITERS, node_body, 0)

    ebase = wid * EDGES_PER_TILE

    def chunk_body(ch, _):
        off = ebase + ch * ECHUNK
        pltpu.sync_copy(qg_hbm.at[pl.ds(off, ECHUNK)], qg_buf)
        pltpu.sync_copy(qt_hbm.at[pl.ds(off, ECHUNK)], qt_buf)
        pltpu.sync_copy(es_hbm.at[pl.ds(off, ECHUNK)], es_buf)

        def vec_body(j, _2):
            for u in range(EUNROLL):
                sl = pl.ds((j * EUNROLL + u) * L, L)
                d = qg_buf[sl] - qt_buf[sl]
                idx = es_buf[sl] * L + lane
                plsc.addupdate_scatter(etab, [idx], d * d)
            return _2

        lax.fori_loop(0, EVEC_ITERS, vec_body, 0)
        return _

    lax.fori_loop(0, N_ECHUNK, chunk_body, 0)

    _reduce_table(etab, partials, 0)
    _reduce_table(ntab, partials, NUM_SEG)
    _reduce_table(ctab, partials, 2 * NUM_SEG)

    pltpu.sync_copy(partials, out_hbm.at[wid])


_SC_SCRATCH = [
    pltpu.VMEM((ECHUNK,), jnp.float32),
    pltpu.VMEM((ECHUNK,), jnp.float32),
    pltpu.VMEM((ECHUNK,), jnp.int32),
    pltpu.VMEM((3, NODES_PER_TILE), jnp.float32),
    pltpu.VMEM((3, NODES_PER_TILE), jnp.float32),
    pltpu.VMEM((NODES_PER_TILE,), jnp.int32),
    pltpu.VMEM((TAB_ROWS * L,), jnp.float32),
    pltpu.VMEM((TAB_ROWS * L,), jnp.float32),
    pltpu.VMEM((TAB_ROWS * L,), jnp.float32),
    pltpu.VMEM((3 * NUM_SEG,), jnp.float32),
]

_sc_partials = pl.kernel(
    _sc_body,
    out_type=jax.ShapeDtypeStruct((NW, 3 * NUM_SEG), jnp.float32),
    mesh=plsc.VectorSubcoreMesh(core_axis_name="c", subcore_axis_name="s"),
    scratch_types=_SC_SCRATCH,
    compiler_params=pltpu.CompilerParams(needs_layout_passes=False),
)


def _fin_body(p_ref, o_ref):
    p = p_ref[...]
    col = jnp.sum(p, axis=0, keepdims=True)
    e = col[:, 0:NUM_SEG]
    n = col[:, NUM_SEG:2 * NUM_SEG]
    c = col[:, 2 * NUM_SEG:3 * NUM_SEG]
    rmsd_m = jnp.sum(jnp.sqrt(n / jnp.maximum(c, 1.0))) / NUM_SEG
    norm_m = jnp.sum(jnp.sqrt(e)) / NUM_SEG
    lanes = lax.broadcasted_iota(jnp.int32, (1, 128), 1)
    o_ref[...] = jnp.where(lanes == 0, rmsd_m,
                           jnp.where(lanes == 1, norm_m, 0.0))


_finalize = pl.pallas_call(
    _fin_body,
    out_shape=jax.ShapeDtypeStruct((1, 128), jnp.float32),
)


@jax.jit
def kernel(x_gen, x_true, node_seg, q_gen, q_true, edge_seg):
    es = edge_seg.astype(jnp.int32)
    ns = node_seg.astype(jnp.int32)
    pad = NODES_PAD - N_NODES
    xg = jnp.pad(x_gen.T, ((0, 0), (0, pad)))
    xt = jnp.pad(x_true.T, ((0, 0), (0, pad)))
    nsp = jnp.pad(ns, (0, pad), constant_values=NUM_SEG)
    partials = _sc_partials(q_gen, q_true, es, xg, xt, nsp)
    out = _finalize(partials)
    return out[0, :2]
